# baseline (speedup 1.0000x reference)
import jax
import jax.numpy as jnp
from jax import lax
from jax.experimental import pallas as pl
from jax.experimental.pallas import tpu as pltpu
from jax.experimental.pallas import tpu_sc as plsc

B = 16384
F = 26
D = 32
NC, NS, L = 2, 16, 16
NW = NC * NS
BPW = B // NW
CHUNK_B = 64
NCHUNK = BPW // CHUNK_B
HALVES = D // L


def _body(x_hbm, table_hbm, bias_hbm, out_hbm, idx_v, rows_v, bias_v, sem):
    wid = lax.axis_index("s") * NC + lax.axis_index("c")
    b0 = wid * BPW

    pltpu.sync_copy(bias_hbm, bias_v)
    bvecs = [bias_v[f, pl.ds(h * L, L)] for f in range(F) for h in range(HALVES)]

    pltpu.sync_copy(x_hbm.at[pl.ds(b0, BPW)], idx_v)

    for c in range(NCHUNK):
        cb = b0 + c * CHUNK_B

        def fire(b, carry):
            pltpu.async_copy(table_hbm.at[idx_v.at[c * CHUNK_B + b]],
                             rows_v.at[b], sem)
            return carry

        lax.fori_loop(0, CHUNK_B, fire, 0)
        pltpu.make_async_copy(out_hbm.at[pl.ds(cb, CHUNK_B)], rows_v, sem).wait()

        def add_bias(b, carry):
            for f in range(F):
                for h in range(HALVES):
                    sl = pl.ds(h * L, L)
                    rows_v[b, f, sl] = rows_v[b, f, sl] + bvecs[f * HALVES + h]
            return carry

        lax.fori_loop(0, CHUNK_B, add_bias, 0)

        pltpu.sync_copy(rows_v, out_hbm.at[pl.ds(cb, CHUNK_B)])


def kernel(X, table, bias):
    mesh = plsc.VectorSubcoreMesh(core_axis_name="c", subcore_axis_name="s")
    f = pl.kernel(
        _body,
        out_type=jax.ShapeDtypeStruct((B, F, D), jnp.float32),
        mesh=mesh,
        compiler_params=pltpu.CompilerParams(use_tc_tiling_on_sc=False),
        scratch_types=[
            pltpu.VMEM((BPW, F), jnp.int32),
            pltpu.VMEM((CHUNK_B, F, D), jnp.float32),
            pltpu.VMEM((F, D), jnp.float32),
            pltpu.SemaphoreType.DMA,
        ],
    )
    return f(X, table, bias)


# speedup vs baseline: 1.4078x; 1.4078x over previous
import jax
import jax.numpy as jnp
from jax import lax
from jax.experimental import pallas as pl
from jax.experimental.pallas import tpu as pltpu
from jax.experimental.pallas import tpu_sc as plsc

B = 16384
F = 26
D = 32
NC, NS, L = 2, 16, 16
NW = NC * NS
BPW = B // NW
CHUNK_B = 32
NCHUNK = BPW // CHUNK_B
CR = CHUNK_B * F
G = 8
NG = CHUNK_B // G
HALVES = D // L


def _body(x_hbm, table_hbm, bias_hbm, out_hbm,
          xi_v, stage_v, bias_v, gsem, ssem):
    wid = lax.axis_index("s") * NC + lax.axis_index("c")

    pltpu.sync_copy(bias_hbm, bias_v)
    bvecs = [bias_v[f, pl.ds(h * L, L)] for f in range(F) for h in range(HALVES)]

    def chunk(c, carry):
        cb = wid * BPW + c * CHUNK_B

        pltpu.sync_copy(x_hbm.at[pl.ds(cb, CHUNK_B)], xi_v)

        def fetch_row(bb, carry2):
            v0 = xi_v[bb, pl.ds(0, L)]
            v1 = xi_v[bb, pl.ds(L, L)]
            for f in range(F):
                i = (v0 if f < L else v1)[f % L]
                pltpu.async_copy(table_hbm.at[pl.ds(i, 1), :],
                                 stage_v.at[pl.ds(bb * F + f, 1)], gsem)
            return carry2

        lax.fori_loop(0, CHUNK_B, fetch_row, 0)

        pltpu.make_async_copy(table_hbm.at[pl.ds(0, CR), :], stage_v,
                              gsem).wait()

        def add_bias(b, carry2):
            for f in range(F):
                for h in range(HALVES):
                    sl = pl.ds(h * L, L)
                    r = b * F + f
                    stage_v[r, sl] = stage_v[r, sl] + bvecs[f * HALVES + h]
            return carry2

        lax.fori_loop(0, CHUNK_B, add_bias, 0)

        def writeout(b, carry2):
            pltpu.async_copy(stage_v.at[pl.ds(b * F, F)], out_hbm.at[cb + b],
                             ssem)
            return carry2

        lax.fori_loop(0, CHUNK_B, writeout, 0)
        pltpu.make_async_copy(table_hbm.at[pl.ds(0, CR), :], stage_v,
                              ssem).wait()
        return carry

    lax.fori_loop(0, NCHUNK, chunk, 0)


def kernel(X, table, bias):
    mesh = plsc.VectorSubcoreMesh(core_axis_name="c", subcore_axis_name="s")
    bias128 = jnp.pad(bias, ((0, 6), (0, 128 - D)))
    x128 = jnp.pad(X, ((0, 0), (0, 128 - F)))
    f = pl.kernel(
        _body,
        out_type=jax.ShapeDtypeStruct((B, F, D), jnp.float32),
        mesh=mesh,
        compiler_params=pltpu.CompilerParams(use_tc_tiling_on_sc=True),
        scratch_types=[
            pltpu.VMEM((CHUNK_B, 128), jnp.int32),
            pltpu.VMEM((CR, D), jnp.float32),
            pltpu.VMEM((32, 128), jnp.float32),
            pltpu.SemaphoreType.DMA,
            pltpu.SemaphoreType.DMA,
        ],
    )
    return f(x128, table, bias128)
